# Initial kernel scaffold; baseline (speedup 1.0000x reference)
import jax
import jax.numpy as jnp
from jax.experimental import pallas as pl


def kernel(x, global_label, weight):
    raise NotImplementedError("write your pallas kernel here")


# speedup vs baseline: 1.8011x; 1.8011x over previous
import jax
import jax.numpy as jnp
from jax.experimental import pallas as pl

BATCH = 64
IN_FEATURES = 512
OUT_FEATURES = 100000
BLOCK_N = 4096


def _fused_norm_matmul_kernel(x_ref, w_ref, out_ref):
    w = w_ref[...]
    ssq = jnp.sum(w * w, axis=1)
    inv = 1.0 / jnp.maximum(jnp.sqrt(ssq), 1e-12)
    acc = jax.lax.dot_general(
        x_ref[...], w,
        dimension_numbers=(((1,), (1,)), ((), ())),
        preferred_element_type=jnp.float32,
    )
    out_ref[...] = acc * inv[None, :]


def kernel(x, global_label, weight):
    del global_label
    grid = pl.cdiv(OUT_FEATURES, BLOCK_N)
    return pl.pallas_call(
        _fused_norm_matmul_kernel,
        grid=(grid,),
        in_specs=[
            pl.BlockSpec((BATCH, IN_FEATURES), lambda i: (0, 0)),
            pl.BlockSpec((BLOCK_N, IN_FEATURES), lambda i: (i, 0)),
        ],
        out_specs=pl.BlockSpec((BATCH, BLOCK_N), lambda i: (0, i)),
        out_shape=jax.ShapeDtypeStruct((BATCH, OUT_FEATURES), jnp.float32),
    )(x, weight)
